# baseline (speedup 1.0000x reference)
import functools

import jax
import jax.numpy as jnp
from jax import lax
from jax.experimental import pallas as pl
from jax.experimental.pallas import tpu as pltpu
from jax.experimental.pallas import tpu_sc as plsc

B = 4
N = 50000
NPAD = 50176
NB = 12544
R = 32
R3 = R * R * R
H = 32
H2 = 64
W = 6272
NWIN = NPAD // W
GRP = W // 16
NEG = -3.4e38


def _mm(a, b):
    return lax.dot_general(a, b, (((1,), (0,)), ((), ())),
                           preferred_element_type=jnp.float32)


def _pad_mask(nb_cols):
    n = pl.program_id(1)
    col = lax.broadcasted_iota(jnp.int32, (H, nb_cols), 1) + n * nb_cols
    return col >= N


def _stage_a_body(pT_ref, w64_ref, b64_ref, w0a_ref, w0b_ref, b0_ref,
                  w1_ref, b1_ref, wsa_ref, wsb_ref, idx_ref, net_ref):
    x = pT_ref[0]
    pn = x + 1.0
    pn = jnp.where(pn >= 2.0, 2.0 - 0.0001, pn)
    pn = jnp.where(pn < 0.0, 0.0, pn)
    xq = ((pn / 2.0) * jnp.float32(R)).astype(jnp.int32)
    idx = xq[0:1, :] + R * xq[1:2, :] + (R * R) * xq[2:3, :]
    idx_ref[0] = idx

    net64 = _mm(w64_ref[...], x) + b64_ref[...]
    rx = jnp.maximum(net64, 0.0)
    h = _mm(w0a_ref[...], rx[:H, :]) + _mm(w0b_ref[...], rx[H:, :]) + b0_ref[...]
    d = _mm(w1_ref[...], jnp.maximum(h, 0.0)) + b1_ref[...]
    s = _mm(wsa_ref[...], net64[:H, :]) + _mm(wsb_ref[...], net64[H:, :])
    net_ref[0] = jnp.where(_pad_mask(NB), NEG, s + d)


def _stage_b_body(net_in_ref, pool_ref, w0a_ref, w0b_ref, b0_ref,
                  w1_ref, b1_ref, wsa_ref, wsb_ref, fc_ref, fcb_ref,
                  net_out_ref, *, final):
    a = net_in_ref[0]
    p = pool_ref[0]
    ra = jnp.maximum(a, 0.0)
    rp = jnp.maximum(p, 0.0)
    h = _mm(w0a_ref[...], ra) + _mm(w0b_ref[...], rp) + b0_ref[...]
    d = _mm(w1_ref[...], jnp.maximum(h, 0.0)) + b1_ref[...]
    out = _mm(wsa_ref[...], a) + _mm(wsb_ref[...], p) + d
    if final:
        out = _mm(fc_ref[...], out) + fcb_ref[...]
    net_out_ref[0] = jnp.where(_pad_mask(NB), NEG, out)


_FULL = lambda shape: pl.BlockSpec(shape, lambda b, n: (0,) * len(shape))


def _tc_stage_a(pT, w64, b64, w0a, w0b, b0, w1, b1, wsa, wsb):
    grid = (B, NPAD // NB)
    return pl.pallas_call(
        _stage_a_body,
        grid=grid,
        in_specs=[
            pl.BlockSpec((1, 3, NB), lambda b, n: (b, 0, n)),
            _FULL((H2, 3)), _FULL((H2, 1)),
            _FULL((H, H)), _FULL((H, H)), _FULL((H, 1)),
            _FULL((H, H)), _FULL((H, 1)),
            _FULL((H, H)), _FULL((H, H)),
        ],
        out_specs=[
            pl.BlockSpec((1, 1, NB), lambda b, n: (b, 0, n)),
            pl.BlockSpec((1, H, NB), lambda b, n: (b, 0, n)),
        ],
        out_shape=[
            jax.ShapeDtypeStruct((B, 1, NPAD), jnp.int32),
            jax.ShapeDtypeStruct((B, H, NPAD), jnp.float32),
        ],
    )(pT, w64, b64, w0a, w0b, b0, w1, b1, wsa, wsb)


def _tc_stage_b(net, pooled, w0a, w0b, b0, w1, b1, wsa, wsb, fc, fcb, final):
    grid = (B, NPAD // NB)
    return pl.pallas_call(
        functools.partial(_stage_b_body, final=final),
        grid=grid,
        in_specs=[
            pl.BlockSpec((1, H, NB), lambda b, n: (b, 0, n)),
            pl.BlockSpec((1, H, NB), lambda b, n: (b, 0, n)),
            _FULL((H, H)), _FULL((H, H)), _FULL((H, 1)),
            _FULL((H, H)), _FULL((H, 1)),
            _FULL((H, H)), _FULL((H, H)),
            _FULL((H, H)), _FULL((H, 1)),
        ],
        out_specs=pl.BlockSpec((1, H, NB), lambda b, n: (b, 0, n)),
        out_shape=jax.ShapeDtypeStruct((B, H, NPAD), jnp.float32),
    )(net, pooled, w0a, w0b, b0, w1, b1, wsa, wsb, fc, fcb)


_NC = 2
_NS = 16


def _grid_init(g_ref):
    neg = jnp.full((16,), -jnp.inf, dtype=jnp.float32)

    @plsc.parallel_loop(0, R3 // 16, unroll=8)
    def _(i):
        g_ref[pl.ds(i * 16, 16)] = neg


def _scatter_window(g_ref, idx_ref, fwin_ref, woff):
    @plsc.parallel_loop(0, GRP, unroll=4)
    def _(i):
        vi = idx_ref[pl.ds(woff + i * 16, 16)]
        fv = fwin_ref[pl.ds(i * 16, 16)]
        cur = plsc.load_gather(g_ref, [vi])
        plsc.store_scatter(g_ref, [vi], jnp.maximum(cur, fv))


def _fixup_window(g_ref, idx_ref, fwin_ref, flg_ref, woff):
    def check(i, a):
        vi = idx_ref[pl.ds(woff + i * 16, 16)]
        fv = fwin_ref[pl.ds(i * 16, 16)]
        need = plsc.load_gather(g_ref, [vi]) < fv
        f = jnp.where(need, 1, 0)
        flg_ref[pl.ds(i * 16, 16)] = f
        return a | f

    dirty = plsc.parallel_loop(
        0, GRP, unroll=4, carry=jnp.zeros((16,), jnp.int32))(check)

    @pl.when(jnp.any(dirty > 0))
    def _():
        def body(i, carry):
            f = flg_ref[pl.ds(i * 16, 16)]

            @pl.when(jnp.any(f > 0))
            def _():
                vi = idx_ref[pl.ds(woff + i * 16, 16)]
                fv = fwin_ref[pl.ds(i * 16, 16)]

                def w_cond(nd):
                    return jnp.any(nd)

                def w_body(nd):
                    c2 = plsc.load_gather(g_ref, [vi])
                    plsc.store_scatter(g_ref, [vi],
                                       jnp.maximum(c2, fv), mask=nd)
                    return plsc.load_gather(g_ref, [vi]) < fv

                lax.while_loop(w_cond, w_body, f > 0)
            return carry

        lax.fori_loop(0, GRP, body, 0)


def _gather_window(g_ref, idx_ref, owin_ref, woff):
    @plsc.parallel_loop(0, GRP, unroll=4)
    def _(i):
        vi = idx_ref[pl.ds(woff + i * 16, 16)]
        owin_ref[pl.ds(i * 16, 16)] = plsc.load_gather(g_ref, [vi])


def _load_idx_staggered(idx_hbm, idxv, sidx, b, wid):
    copies = []
    for j in range(NWIN):
        ck = lax.rem(wid + j, NWIN)
        copies.append(pltpu.async_copy(
            idx_hbm.at[b, 0, pl.ds(ck * W, W)],
            idxv.at[pl.ds(ck * W, W)], sidx))
    return copies


def _scatter_phase(idx_hbm, net_hbm, g_ref, idxv, f0, f1, flg, sidx,
                   sf0, sf1, b, wid, idx_copies=None):
    if idx_copies is None:
        idx_copies = _load_idx_staggered(idx_hbm, idxv, sidx, b, wid)
    _grid_init(g_ref)
    for c in idx_copies:
        c.wait()
    fb, sfb = [f0, f1], [sf0, sf1]
    cps = [pltpu.async_copy(net_hbm.at[b, wid, pl.ds(0, W)], f0, sf0), None]
    for w in range(NWIN):
        cur = w % 2
        if w + 1 < NWIN:
            nxt = (w + 1) % 2
            cps[nxt] = pltpu.async_copy(
                net_hbm.at[b, wid, pl.ds((w + 1) * W, W)], fb[nxt], sfb[nxt])
        cps[cur].wait()
        _scatter_window(g_ref, idxv, fb[cur], w * W)
        _fixup_window(g_ref, idxv, fb[cur], flg, w * W)


def _sc_pool_body(idx_hbm, net_hbm, out_hbm, g_ref, idxv, f0, f1, o0, o1,
                  flg, sidx, sf0, sf1, so0, so1):
    wid = lax.axis_index("s") * _NC + lax.axis_index("c")
    ob, sob = [o0, o1], [so0, so1]
    pre = None
    for b in range(B):
        _scatter_phase(idx_hbm, net_hbm, g_ref, idxv, f0, f1, flg,
                       sidx, sf0, sf1, b, wid, idx_copies=pre)
        pre = [] if b + 1 < B else None
        ocp = [None, None]
        for w in range(NWIN):
            cur = w % 2
            if ocp[cur] is not None:
                ocp[cur].wait()
            _gather_window(g_ref, idxv, ob[cur], w * W)
            if pre is not None:
                pre.append(pltpu.async_copy(
                    idx_hbm.at[b + 1, 0, pl.ds(w * W, W)],
                    idxv.at[pl.ds(w * W, W)], sidx))
            ocp[cur] = pltpu.async_copy(
                ob[cur], out_hbm.at[b, wid, pl.ds(w * W, W)], sob[cur])
        ocp[0].wait()
        ocp[1].wait()


def _sc_final_body(idx_hbm, net_hbm, out_hbm, g_ref, idxv, f0, f1, o0, o1,
                   flg, sidx, sf0, sf1, so0, so1):
    wid = lax.axis_index("s") * _NC + lax.axis_index("c")
    ob, sob = [o0, o1], [so0, so1]
    CH = 4096
    pre = None
    for b in range(B):
        _scatter_phase(idx_hbm, net_hbm, g_ref, idxv, f0, f1, flg,
                       sidx, sf0, sf1, b, wid, idx_copies=pre)
        pre = (_load_idx_staggered(idx_hbm, idxv, sidx, b + 1, wid)
               if b + 1 < B else None)
        ocp = [None, None]
        for c in range(R3 // CH):
            cur = c % 2
            if ocp[cur] is not None:
                ocp[cur].wait()

            @plsc.parallel_loop(0, CH // 16, unroll=8)
            def _(i, _c=c, _cur=cur):
                g = g_ref[pl.ds(_c * CH + i * 16, 16)]
                ob[_cur][pl.ds(i * 16, 16)] = jnp.maximum(g, 0.0)
            ocp[cur] = pltpu.async_copy(
                ob[cur].at[pl.ds(0, CH)],
                out_hbm.at[b, wid, pl.ds(c * CH, CH)], sob[cur])
        ocp[0].wait()
        ocp[1].wait()


@functools.cache
def _sc_kernels():
    mesh = plsc.VectorSubcoreMesh(core_axis_name="c", subcore_axis_name="s",
                                  num_cores=_NC, num_subcores=_NS)
    scratch = [
        pltpu.VMEM((R3,), jnp.float32),
        pltpu.VMEM((NPAD,), jnp.int32),
        pltpu.VMEM((W,), jnp.float32),
        pltpu.VMEM((W,), jnp.float32),
        pltpu.VMEM((W,), jnp.float32),
        pltpu.VMEM((W,), jnp.float32),
        pltpu.VMEM((W,), jnp.int32),
        pltpu.SemaphoreType.DMA,
        pltpu.SemaphoreType.DMA,
        pltpu.SemaphoreType.DMA,
        pltpu.SemaphoreType.DMA,
        pltpu.SemaphoreType.DMA,
    ]
    params = pltpu.CompilerParams(needs_layout_passes=False)
    sc_pool = pl.kernel(
        _sc_pool_body,
        out_type=jax.ShapeDtypeStruct((B, H, NPAD), jnp.float32),
        mesh=mesh,
        scratch_types=scratch,
        compiler_params=params,
    )
    sc_final = pl.kernel(
        _sc_final_body,
        out_type=jax.ShapeDtypeStruct((B, H, R3), jnp.float32),
        mesh=mesh,
        scratch_types=scratch,
        compiler_params=params,
    )
    return sc_pool, sc_final


def kernel(p, fc_pos_W, fc_pos_b, blocks_W0, blocks_b0, blocks_W1,
           blocks_b1, blocks_Ws, fc_c_W, fc_c_b):
    f32 = jnp.float32
    pT = jnp.transpose(p, (0, 2, 1))
    pT = jnp.pad(pT, ((0, 0), (0, 0), (0, NPAD - N)))

    w64 = fc_pos_W.T
    b64 = fc_pos_b.reshape(H2, 1).astype(f32)
    w0 = jnp.transpose(blocks_W0, (0, 2, 1))
    w0a, w0b = w0[:, :, :H], w0[:, :, H:]
    b0 = blocks_b0.reshape(-1, H, 1)
    w1 = jnp.transpose(blocks_W1, (0, 2, 1))
    b1 = blocks_b1.reshape(-1, H, 1)
    ws = jnp.transpose(blocks_Ws, (0, 2, 1))
    wsa, wsb = ws[:, :, :H], ws[:, :, H:]
    fc = fc_c_W.T
    fcb = fc_c_b.reshape(H, 1)

    sc_pool, sc_final = _sc_kernels()
    idx, net = _tc_stage_a(pT, w64, b64, w0a[0], w0b[0], b0[0],
                           w1[0], b1[0], wsa[0], wsb[0])
    for i in range(1, 5):
        pooled = sc_pool(idx, net)
        net = _tc_stage_b(net, pooled, w0a[i], w0b[i], b0[i],
                          w1[i], b1[i], wsa[i], wsb[i], fc, fcb,
                          final=(i == 4))
    grid = sc_final(idx, net)
    return grid.reshape(B, H, R, R, R)


# speedup vs baseline: 1.7777x; 1.7777x over previous
import functools

import jax
import jax.numpy as jnp
from jax import lax
from jax.experimental import pallas as pl
from jax.experimental.pallas import tpu as pltpu
from jax.experimental.pallas import tpu_sc as plsc

B = 4
N = 50000
NPAD = 50176
NB = 12544
R = 32
R3 = R * R * R
H = 32
H2 = 64
W = 6272
NWIN = NPAD // W
GRP = W // 16
NEG = -3.4e38


def _mm(a, b):
    return lax.dot_general(a, b, (((1,), (0,)), ((), ())),
                           preferred_element_type=jnp.float32)


def _pad_mask(nb_cols):
    n = pl.program_id(1)
    col = lax.broadcasted_iota(jnp.int32, (H, nb_cols), 1) + n * nb_cols
    return col >= N


def _stage_a_body(pT_ref, w64_ref, b64_ref, w0a_ref, w0b_ref, b0_ref,
                  w1_ref, b1_ref, wsa_ref, wsb_ref, idx_ref, net_ref):
    x = pT_ref[0]
    pn = x + 1.0
    pn = jnp.where(pn >= 2.0, 2.0 - 0.0001, pn)
    pn = jnp.where(pn < 0.0, 0.0, pn)
    xq = ((pn / 2.0) * jnp.float32(R)).astype(jnp.int32)
    idx = xq[0:1, :] + R * xq[1:2, :] + (R * R) * xq[2:3, :]
    idx_ref[0] = idx

    net64 = _mm(w64_ref[...], x) + b64_ref[...]
    rx = jnp.maximum(net64, 0.0)
    h = _mm(w0a_ref[...], rx[:H, :]) + _mm(w0b_ref[...], rx[H:, :]) + b0_ref[...]
    d = _mm(w1_ref[...], jnp.maximum(h, 0.0)) + b1_ref[...]
    s = _mm(wsa_ref[...], net64[:H, :]) + _mm(wsb_ref[...], net64[H:, :])
    net_ref[0] = jnp.where(_pad_mask(NB), NEG, s + d)


def _stage_b_body(net_in_ref, pool_ref, w0a_ref, w0b_ref, b0_ref,
                  w1_ref, b1_ref, wsa_ref, wsb_ref, fc_ref, fcb_ref,
                  net_out_ref, *, final):
    a = net_in_ref[0]
    p = pool_ref[0]
    ra = jnp.maximum(a, 0.0)
    rp = jnp.maximum(p, 0.0)
    h = _mm(w0a_ref[...], ra) + _mm(w0b_ref[...], rp) + b0_ref[...]
    d = _mm(w1_ref[...], jnp.maximum(h, 0.0)) + b1_ref[...]
    out = _mm(wsa_ref[...], a) + _mm(wsb_ref[...], p) + d
    if final:
        out = _mm(fc_ref[...], out) + fcb_ref[...]
    net_out_ref[0] = jnp.where(_pad_mask(NB), NEG, out)


_FULL = lambda shape: pl.BlockSpec(shape, lambda b, n: (0,) * len(shape))


def _tc_stage_a(pT, w64, b64, w0a, w0b, b0, w1, b1, wsa, wsb):
    grid = (B, NPAD // NB)
    return pl.pallas_call(
        _stage_a_body,
        grid=grid,
        in_specs=[
            pl.BlockSpec((1, 3, NB), lambda b, n: (b, 0, n)),
            _FULL((H2, 3)), _FULL((H2, 1)),
            _FULL((H, H)), _FULL((H, H)), _FULL((H, 1)),
            _FULL((H, H)), _FULL((H, 1)),
            _FULL((H, H)), _FULL((H, H)),
        ],
        out_specs=[
            pl.BlockSpec((1, 1, NB), lambda b, n: (b, 0, n)),
            pl.BlockSpec((1, H, NB), lambda b, n: (b, 0, n)),
        ],
        out_shape=[
            jax.ShapeDtypeStruct((B, 1, NPAD), jnp.int32),
            jax.ShapeDtypeStruct((B, H, NPAD), jnp.float32),
        ],
    )(pT, w64, b64, w0a, w0b, b0, w1, b1, wsa, wsb)


def _tc_stage_b(net, pooled, w0a, w0b, b0, w1, b1, wsa, wsb, fc, fcb, final):
    grid = (B, NPAD // NB)
    return pl.pallas_call(
        functools.partial(_stage_b_body, final=final),
        grid=grid,
        in_specs=[
            pl.BlockSpec((1, H, NB), lambda b, n: (b, 0, n)),
            pl.BlockSpec((1, H, NB), lambda b, n: (b, 0, n)),
            _FULL((H, H)), _FULL((H, H)), _FULL((H, 1)),
            _FULL((H, H)), _FULL((H, 1)),
            _FULL((H, H)), _FULL((H, H)),
            _FULL((H, H)), _FULL((H, 1)),
        ],
        out_specs=pl.BlockSpec((1, H, NB), lambda b, n: (b, 0, n)),
        out_shape=jax.ShapeDtypeStruct((B, H, NPAD), jnp.float32),
    )(net, pooled, w0a, w0b, b0, w1, b1, wsa, wsb, fc, fcb)


_NC = 2
_NS = 16


def _grid_init(g_ref):
    neg = jnp.full((16,), -jnp.inf, dtype=jnp.float32)

    @plsc.parallel_loop(0, R3 // 16, unroll=8)
    def _(i):
        g_ref[pl.ds(i * 16, 16)] = neg


def _scatter_window(g_ref, idx_ref, fwin_ref, woff):
    @plsc.parallel_loop(0, GRP, unroll=4)
    def _(i):
        vi = idx_ref[pl.ds(woff + i * 16, 16)]
        fv = fwin_ref[pl.ds(i * 16, 16)]
        cur = plsc.load_gather(g_ref, [vi])
        plsc.store_scatter(g_ref, [vi], jnp.maximum(cur, fv))


def _fixup_window(g_ref, idx_ref, fwin_ref, flg_ref, woff):
    del flg_ref

    def body(i, carry):
        base = i * 64
        vis, fvs, needs = [], [], []
        for u in range(4):
            off = base + u * 16
            vi = idx_ref[pl.ds(woff + off, 16)]
            fv = fwin_ref[pl.ds(off, 16)]
            vis.append(vi)
            fvs.append(fv)
            needs.append(plsc.load_gather(g_ref, [vi]) < fv)
        acc = (needs[0] | needs[1]) | (needs[2] | needs[3])

        @pl.when(jnp.any(acc))
        def _():
            for u in range(4):
                vi, fv = vis[u], fvs[u]

                def w_cond(nd):
                    return jnp.any(nd)

                def w_body(nd, vi=vi, fv=fv):
                    c2 = plsc.load_gather(g_ref, [vi])
                    plsc.store_scatter(g_ref, [vi], jnp.maximum(c2, fv),
                                       mask=nd)
                    return plsc.load_gather(g_ref, [vi]) < fv

                lax.while_loop(w_cond, w_body, needs[u])
        return carry

    lax.fori_loop(0, GRP // 4, body, 0)


def _gather_window(g_ref, idx_ref, owin_ref, woff):
    @plsc.parallel_loop(0, GRP, unroll=4)
    def _(i):
        vi = idx_ref[pl.ds(woff + i * 16, 16)]
        owin_ref[pl.ds(i * 16, 16)] = plsc.load_gather(g_ref, [vi])


def _load_idx_staggered(idx_hbm, idxv, sidx, b, wid):
    copies = []
    for j in range(NWIN):
        ck = lax.rem(wid + j, NWIN)
        copies.append(pltpu.async_copy(
            idx_hbm.at[b, 0, pl.ds(ck * W, W)],
            idxv.at[pl.ds(ck * W, W)], sidx))
    return copies


def _scatter_phase(idx_hbm, net_hbm, g_ref, idxv, f0, f1, flg, sidx,
                   sf0, sf1, b, wid, idx_copies=None):
    if idx_copies is None:
        idx_copies = _load_idx_staggered(idx_hbm, idxv, sidx, b, wid)
    _grid_init(g_ref)
    for c in idx_copies:
        c.wait()
    fb, sfb = [f0, f1], [sf0, sf1]
    cps = [pltpu.async_copy(net_hbm.at[b, wid, pl.ds(0, W)], f0, sf0), None]
    for w in range(NWIN):
        cur = w % 2
        if w + 1 < NWIN:
            nxt = (w + 1) % 2
            cps[nxt] = pltpu.async_copy(
                net_hbm.at[b, wid, pl.ds((w + 1) * W, W)], fb[nxt], sfb[nxt])
        cps[cur].wait()
        _scatter_window(g_ref, idxv, fb[cur], w * W)
        _fixup_window(g_ref, idxv, fb[cur], flg, w * W)


def _sc_pool_body(idx_hbm, net_hbm, out_hbm, g_ref, idxv, f0, f1, o0, o1,
                  flg, sidx, sf0, sf1, so0, so1):
    wid = lax.axis_index("s") * _NC + lax.axis_index("c")
    ob, sob = [o0, o1], [so0, so1]
    pre = None
    for b in range(B):
        _scatter_phase(idx_hbm, net_hbm, g_ref, idxv, f0, f1, flg,
                       sidx, sf0, sf1, b, wid, idx_copies=pre)
        pre = [] if b + 1 < B else None
        ocp = [None, None]
        for w in range(NWIN):
            cur = w % 2
            if ocp[cur] is not None:
                ocp[cur].wait()
            _gather_window(g_ref, idxv, ob[cur], w * W)
            if pre is not None:
                pre.append(pltpu.async_copy(
                    idx_hbm.at[b + 1, 0, pl.ds(w * W, W)],
                    idxv.at[pl.ds(w * W, W)], sidx))
            ocp[cur] = pltpu.async_copy(
                ob[cur], out_hbm.at[b, wid, pl.ds(w * W, W)], sob[cur])
        ocp[0].wait()
        ocp[1].wait()


def _sc_final_body(idx_hbm, net_hbm, out_hbm, g_ref, idxv, f0, f1, o0, o1,
                   flg, sidx, sf0, sf1, so0, so1):
    wid = lax.axis_index("s") * _NC + lax.axis_index("c")
    ob, sob = [o0, o1], [so0, so1]
    CH = 4096
    pre = None
    for b in range(B):
        _scatter_phase(idx_hbm, net_hbm, g_ref, idxv, f0, f1, flg,
                       sidx, sf0, sf1, b, wid, idx_copies=pre)
        pre = (_load_idx_staggered(idx_hbm, idxv, sidx, b + 1, wid)
               if b + 1 < B else None)
        ocp = [None, None]
        for c in range(R3 // CH):
            cur = c % 2
            if ocp[cur] is not None:
                ocp[cur].wait()

            @plsc.parallel_loop(0, CH // 16, unroll=8)
            def _(i, _c=c, _cur=cur):
                g = g_ref[pl.ds(_c * CH + i * 16, 16)]
                ob[_cur][pl.ds(i * 16, 16)] = jnp.maximum(g, 0.0)
            ocp[cur] = pltpu.async_copy(
                ob[cur].at[pl.ds(0, CH)],
                out_hbm.at[b, wid, pl.ds(c * CH, CH)], sob[cur])
        ocp[0].wait()
        ocp[1].wait()


@functools.cache
def _sc_kernels():
    mesh = plsc.VectorSubcoreMesh(core_axis_name="c", subcore_axis_name="s",
                                  num_cores=_NC, num_subcores=_NS)
    scratch = [
        pltpu.VMEM((R3,), jnp.float32),
        pltpu.VMEM((NPAD,), jnp.int32),
        pltpu.VMEM((W,), jnp.float32),
        pltpu.VMEM((W,), jnp.float32),
        pltpu.VMEM((W,), jnp.float32),
        pltpu.VMEM((W,), jnp.float32),
        pltpu.VMEM((W,), jnp.int32),
        pltpu.SemaphoreType.DMA,
        pltpu.SemaphoreType.DMA,
        pltpu.SemaphoreType.DMA,
        pltpu.SemaphoreType.DMA,
        pltpu.SemaphoreType.DMA,
    ]
    params = pltpu.CompilerParams(needs_layout_passes=False)
    sc_pool = pl.kernel(
        _sc_pool_body,
        out_type=jax.ShapeDtypeStruct((B, H, NPAD), jnp.float32),
        mesh=mesh,
        scratch_types=scratch,
        compiler_params=params,
    )
    sc_final = pl.kernel(
        _sc_final_body,
        out_type=jax.ShapeDtypeStruct((B, H, R3), jnp.float32),
        mesh=mesh,
        scratch_types=scratch,
        compiler_params=params,
    )
    return sc_pool, sc_final


def kernel(p, fc_pos_W, fc_pos_b, blocks_W0, blocks_b0, blocks_W1,
           blocks_b1, blocks_Ws, fc_c_W, fc_c_b):
    f32 = jnp.float32
    pT = jnp.transpose(p, (0, 2, 1))
    pT = jnp.pad(pT, ((0, 0), (0, 0), (0, NPAD - N)))

    w64 = fc_pos_W.T
    b64 = fc_pos_b.reshape(H2, 1).astype(f32)
    w0 = jnp.transpose(blocks_W0, (0, 2, 1))
    w0a, w0b = w0[:, :, :H], w0[:, :, H:]
    b0 = blocks_b0.reshape(-1, H, 1)
    w1 = jnp.transpose(blocks_W1, (0, 2, 1))
    b1 = blocks_b1.reshape(-1, H, 1)
    ws = jnp.transpose(blocks_Ws, (0, 2, 1))
    wsa, wsb = ws[:, :, :H], ws[:, :, H:]
    fc = fc_c_W.T
    fcb = fc_c_b.reshape(H, 1)

    sc_pool, sc_final = _sc_kernels()
    idx, net = _tc_stage_a(pT, w64, b64, w0a[0], w0b[0], b0[0],
                           w1[0], b1[0], wsa[0], wsb[0])
    for i in range(1, 5):
        pooled = sc_pool(idx, net)
        net = _tc_stage_b(net, pooled, w0a[i], w0b[i], b0[i],
                          w1[i], b1[i], wsa[i], wsb[i], fc, fcb,
                          final=(i == 4))
    grid = sc_final(idx, net)
    return grid.reshape(B, H, R, R, R)


# speedup vs baseline: 1.7959x; 1.0102x over previous
import functools

import jax
import jax.numpy as jnp
from jax import lax
from jax.experimental import pallas as pl
from jax.experimental.pallas import tpu as pltpu
from jax.experimental.pallas import tpu_sc as plsc

B = 4
N = 50000
NPAD = 50176
NB = 25088
R = 32
R3 = R * R * R
H = 32
H2 = 64
W = 6272
NWIN = NPAD // W
GRP = W // 16
NEG = -3.4e38


def _mm(a, b):
    return lax.dot_general(a, b, (((1,), (0,)), ((), ())),
                           preferred_element_type=jnp.float32)


def _pad_mask(nb_cols):
    n = pl.program_id(1)
    col = lax.broadcasted_iota(jnp.int32, (H, nb_cols), 1) + n * nb_cols
    return col >= N


def _stage_a_body(pT_ref, w64_ref, b64_ref, w0a_ref, w0b_ref, b0_ref,
                  w1_ref, b1_ref, wsa_ref, wsb_ref, idx_ref, net_ref):
    x = pT_ref[0]
    pn = x + 1.0
    pn = jnp.where(pn >= 2.0, 2.0 - 0.0001, pn)
    pn = jnp.where(pn < 0.0, 0.0, pn)
    xq = ((pn / 2.0) * jnp.float32(R)).astype(jnp.int32)
    idx = xq[0:1, :] + R * xq[1:2, :] + (R * R) * xq[2:3, :]
    idx_ref[0] = idx

    net64 = _mm(w64_ref[...], x) + b64_ref[...]
    rx = jnp.maximum(net64, 0.0)
    h = _mm(w0a_ref[...], rx[:H, :]) + _mm(w0b_ref[...], rx[H:, :]) + b0_ref[...]
    d = _mm(w1_ref[...], jnp.maximum(h, 0.0)) + b1_ref[...]
    s = _mm(wsa_ref[...], net64[:H, :]) + _mm(wsb_ref[...], net64[H:, :])
    net_ref[0] = jnp.where(_pad_mask(NB), NEG, s + d)


def _stage_b_body(net_in_ref, pool_ref, w0a_ref, w0b_ref, b0_ref,
                  w1_ref, b1_ref, wsa_ref, wsb_ref, fc_ref, fcb_ref,
                  net_out_ref, *, final):
    a = net_in_ref[0]
    p = pool_ref[0]
    ra = jnp.maximum(a, 0.0)
    rp = jnp.maximum(p, 0.0)
    h = _mm(w0a_ref[...], ra) + _mm(w0b_ref[...], rp) + b0_ref[...]
    d = _mm(w1_ref[...], jnp.maximum(h, 0.0)) + b1_ref[...]
    out = _mm(wsa_ref[...], a) + _mm(wsb_ref[...], p) + d
    if final:
        out = _mm(fc_ref[...], out) + fcb_ref[...]
    net_out_ref[0] = jnp.where(_pad_mask(NB), NEG, out)


_FULL = lambda shape: pl.BlockSpec(shape, lambda b, n: (0,) * len(shape))


def _tc_stage_a(pT, w64, b64, w0a, w0b, b0, w1, b1, wsa, wsb):
    grid = (B, NPAD // NB)
    return pl.pallas_call(
        _stage_a_body,
        grid=grid,
        in_specs=[
            pl.BlockSpec((1, 3, NB), lambda b, n: (b, 0, n)),
            _FULL((H2, 3)), _FULL((H2, 1)),
            _FULL((H, H)), _FULL((H, H)), _FULL((H, 1)),
            _FULL((H, H)), _FULL((H, 1)),
            _FULL((H, H)), _FULL((H, H)),
        ],
        out_specs=[
            pl.BlockSpec((1, 1, NB), lambda b, n: (b, 0, n)),
            pl.BlockSpec((1, H, NB), lambda b, n: (b, 0, n)),
        ],
        out_shape=[
            jax.ShapeDtypeStruct((B, 1, NPAD), jnp.int32),
            jax.ShapeDtypeStruct((B, H, NPAD), jnp.float32),
        ],
    )(pT, w64, b64, w0a, w0b, b0, w1, b1, wsa, wsb)


def _tc_stage_b(net, pooled, w0a, w0b, b0, w1, b1, wsa, wsb, fc, fcb, final):
    grid = (B, NPAD // NB)
    return pl.pallas_call(
        functools.partial(_stage_b_body, final=final),
        grid=grid,
        in_specs=[
            pl.BlockSpec((1, H, NB), lambda b, n: (b, 0, n)),
            pl.BlockSpec((1, H, NB), lambda b, n: (b, 0, n)),
            _FULL((H, H)), _FULL((H, H)), _FULL((H, 1)),
            _FULL((H, H)), _FULL((H, 1)),
            _FULL((H, H)), _FULL((H, H)),
            _FULL((H, H)), _FULL((H, 1)),
        ],
        out_specs=pl.BlockSpec((1, H, NB), lambda b, n: (b, 0, n)),
        out_shape=jax.ShapeDtypeStruct((B, H, NPAD), jnp.float32),
    )(net, pooled, w0a, w0b, b0, w1, b1, wsa, wsb, fc, fcb)


_NC = 2
_NS = 16


def _grid_init(g_ref):
    neg = jnp.full((16,), -jnp.inf, dtype=jnp.float32)

    @plsc.parallel_loop(0, R3 // 16, unroll=8)
    def _(i):
        g_ref[pl.ds(i * 16, 16)] = neg


def _scatter_window(g_ref, idx_ref, fwin_ref, woff):
    @plsc.parallel_loop(0, GRP, unroll=4)
    def _(i):
        vi = idx_ref[pl.ds(woff + i * 16, 16)]
        fv = fwin_ref[pl.ds(i * 16, 16)]
        cur = plsc.load_gather(g_ref, [vi])
        plsc.store_scatter(g_ref, [vi], jnp.maximum(cur, fv))


def _fixup_window(g_ref, idx_ref, fwin_ref, flg_ref, woff):
    del flg_ref

    def body(i, carry):
        base = i * 64
        vis, fvs, needs = [], [], []
        for u in range(4):
            off = base + u * 16
            vi = idx_ref[pl.ds(woff + off, 16)]
            fv = fwin_ref[pl.ds(off, 16)]
            vis.append(vi)
            fvs.append(fv)
            needs.append(plsc.load_gather(g_ref, [vi]) < fv)
        acc = (needs[0] | needs[1]) | (needs[2] | needs[3])

        @pl.when(jnp.any(acc))
        def _():
            for u in range(4):
                vi, fv = vis[u], fvs[u]

                def w_cond(nd):
                    return jnp.any(nd)

                def w_body(nd, vi=vi, fv=fv):
                    c2 = plsc.load_gather(g_ref, [vi])
                    plsc.store_scatter(g_ref, [vi], jnp.maximum(c2, fv),
                                       mask=nd)
                    return plsc.load_gather(g_ref, [vi]) < fv

                lax.while_loop(w_cond, w_body, needs[u])
        return carry

    lax.fori_loop(0, GRP // 4, body, 0)


def _gather_window(g_ref, idx_ref, owin_ref, woff):
    @plsc.parallel_loop(0, GRP, unroll=4)
    def _(i):
        vi = idx_ref[pl.ds(woff + i * 16, 16)]
        owin_ref[pl.ds(i * 16, 16)] = plsc.load_gather(g_ref, [vi])


def _load_idx_staggered(idx_hbm, idxv, sidx, b, wid):
    copies = []
    for j in range(NWIN):
        ck = lax.rem(wid + j, NWIN)
        copies.append(pltpu.async_copy(
            idx_hbm.at[b, 0, pl.ds(ck * W, W)],
            idxv.at[pl.ds(ck * W, W)], sidx))
    return copies


def _scatter_phase(idx_hbm, net_hbm, g_ref, idxv, f0, f1, flg, sidx,
                   sf0, sf1, b, wid, idx_copies=None):
    if idx_copies is None:
        idx_copies = _load_idx_staggered(idx_hbm, idxv, sidx, b, wid)
    _grid_init(g_ref)
    for c in idx_copies:
        c.wait()
    fb, sfb = [f0, f1], [sf0, sf1]
    cps = [pltpu.async_copy(net_hbm.at[b, wid, pl.ds(0, W)], f0, sf0), None]
    for w in range(NWIN):
        cur = w % 2
        if w + 1 < NWIN:
            nxt = (w + 1) % 2
            cps[nxt] = pltpu.async_copy(
                net_hbm.at[b, wid, pl.ds((w + 1) * W, W)], fb[nxt], sfb[nxt])
        cps[cur].wait()
        _scatter_window(g_ref, idxv, fb[cur], w * W)
        _fixup_window(g_ref, idxv, fb[cur], flg, w * W)


def _sc_pool_body(idx_hbm, net_hbm, out_hbm, g_ref, idxv, f0, f1, o0, o1,
                  flg, sidx, sf0, sf1, so0, so1):
    wid = lax.axis_index("s") * _NC + lax.axis_index("c")
    ob, sob = [o0, o1], [so0, so1]
    pre = None
    for b in range(B):
        _scatter_phase(idx_hbm, net_hbm, g_ref, idxv, f0, f1, flg,
                       sidx, sf0, sf1, b, wid, idx_copies=pre)
        pre = [] if b + 1 < B else None
        ocp = [None, None]
        for w in range(NWIN):
            cur = w % 2
            if ocp[cur] is not None:
                ocp[cur].wait()
            _gather_window(g_ref, idxv, ob[cur], w * W)
            if pre is not None:
                pre.append(pltpu.async_copy(
                    idx_hbm.at[b + 1, 0, pl.ds(w * W, W)],
                    idxv.at[pl.ds(w * W, W)], sidx))
            ocp[cur] = pltpu.async_copy(
                ob[cur], out_hbm.at[b, wid, pl.ds(w * W, W)], sob[cur])
        ocp[0].wait()
        ocp[1].wait()


def _sc_final_body(idx_hbm, net_hbm, out_hbm, g_ref, idxv, f0, f1, o0, o1,
                   flg, sidx, sf0, sf1, so0, so1):
    wid = lax.axis_index("s") * _NC + lax.axis_index("c")
    ob, sob = [o0, o1], [so0, so1]
    CH = 4096
    pre = None
    for b in range(B):
        _scatter_phase(idx_hbm, net_hbm, g_ref, idxv, f0, f1, flg,
                       sidx, sf0, sf1, b, wid, idx_copies=pre)
        pre = (_load_idx_staggered(idx_hbm, idxv, sidx, b + 1, wid)
               if b + 1 < B else None)
        ocp = [None, None]
        for c in range(R3 // CH):
            cur = c % 2
            if ocp[cur] is not None:
                ocp[cur].wait()

            @plsc.parallel_loop(0, CH // 16, unroll=8)
            def _(i, _c=c, _cur=cur):
                g = g_ref[pl.ds(_c * CH + i * 16, 16)]
                ob[_cur][pl.ds(i * 16, 16)] = jnp.maximum(g, 0.0)
            ocp[cur] = pltpu.async_copy(
                ob[cur].at[pl.ds(0, CH)],
                out_hbm.at[b, wid, pl.ds(c * CH, CH)], sob[cur])
        ocp[0].wait()
        ocp[1].wait()


@functools.cache
def _sc_kernels():
    mesh = plsc.VectorSubcoreMesh(core_axis_name="c", subcore_axis_name="s",
                                  num_cores=_NC, num_subcores=_NS)
    scratch = [
        pltpu.VMEM((R3,), jnp.float32),
        pltpu.VMEM((NPAD,), jnp.int32),
        pltpu.VMEM((W,), jnp.float32),
        pltpu.VMEM((W,), jnp.float32),
        pltpu.VMEM((W,), jnp.float32),
        pltpu.VMEM((W,), jnp.float32),
        pltpu.VMEM((W,), jnp.int32),
        pltpu.SemaphoreType.DMA,
        pltpu.SemaphoreType.DMA,
        pltpu.SemaphoreType.DMA,
        pltpu.SemaphoreType.DMA,
        pltpu.SemaphoreType.DMA,
    ]
    params = pltpu.CompilerParams(needs_layout_passes=False)
    sc_pool = pl.kernel(
        _sc_pool_body,
        out_type=jax.ShapeDtypeStruct((B, H, NPAD), jnp.float32),
        mesh=mesh,
        scratch_types=scratch,
        compiler_params=params,
    )
    sc_final = pl.kernel(
        _sc_final_body,
        out_type=jax.ShapeDtypeStruct((B, H, R3), jnp.float32),
        mesh=mesh,
        scratch_types=scratch,
        compiler_params=params,
    )
    return sc_pool, sc_final


def kernel(p, fc_pos_W, fc_pos_b, blocks_W0, blocks_b0, blocks_W1,
           blocks_b1, blocks_Ws, fc_c_W, fc_c_b):
    f32 = jnp.float32
    pT = jnp.transpose(p, (0, 2, 1))
    pT = jnp.pad(pT, ((0, 0), (0, 0), (0, NPAD - N)))

    w64 = fc_pos_W.T
    b64 = fc_pos_b.reshape(H2, 1).astype(f32)
    w0 = jnp.transpose(blocks_W0, (0, 2, 1))
    w0a, w0b = w0[:, :, :H], w0[:, :, H:]
    b0 = blocks_b0.reshape(-1, H, 1)
    w1 = jnp.transpose(blocks_W1, (0, 2, 1))
    b1 = blocks_b1.reshape(-1, H, 1)
    ws = jnp.transpose(blocks_Ws, (0, 2, 1))
    wsa, wsb = ws[:, :, :H], ws[:, :, H:]
    fc = fc_c_W.T
    fcb = fc_c_b.reshape(H, 1)

    sc_pool, sc_final = _sc_kernels()
    idx, net = _tc_stage_a(pT, w64, b64, w0a[0], w0b[0], b0[0],
                           w1[0], b1[0], wsa[0], wsb[0])
    for i in range(1, 5):
        pooled = sc_pool(idx, net)
        net = _tc_stage_b(net, pooled, w0a[i], w0b[i], b0[i],
                          w1[i], b1[i], wsa[i], wsb[i], fc, fcb,
                          final=(i == 4))
    grid = sc_final(idx, net)
    return grid.reshape(B, H, R, R, R)
